# baseline (speedup 1.0000x reference)
import functools

import jax
import jax.numpy as jnp
from jax import lax
from jax.experimental import pallas as pl
from jax.experimental.pallas import tpu as pltpu
from jax.experimental.pallas import tpu_sc as plsc

_NC = 2
_NS = 16
_L = 128


def _sc_mesh():
    return plsc.VectorSubcoreMesh(
        core_axis_name="c", subcore_axis_name="s",
        num_cores=_NC, num_subcores=_NS)


_SC_PARAMS = pltpu.CompilerParams(use_tc_tiling_on_sc=False)


def _sc_degrees(srcp, dstp, maskp, n_pad, sl):
    nw, nch, cc = srcp.shape

    @functools.partial(
        pl.kernel,
        out_type=jax.ShapeDtypeStruct((_NC * 2 * n_pad,), jnp.float32),
        mesh=_sc_mesh(),
        compiler_params=_SC_PARAMS,
        scratch_types=[
            pltpu.VMEM((nch, cc), jnp.int32),
            pltpu.VMEM((nch, cc), jnp.float32),
            pltpu.VMEM((sl,), jnp.float32),
            pltpu.VMEM_SHARED((n_pad,), jnp.float32),
            pltpu.VMEM_SHARED((n_pad,), jnp.float32),
        ],
    )
    def deg_kernel(srcp_hbm, dstp_hbm, maskp_hbm, out_hbm,
                   idx_v, val_v, stage_v, dego_sh, degi_sh):
        c = lax.axis_index("c")
        s = lax.axis_index("s")
        w = c * _NS + s
        off = s * sl

        def zbody(k, carry):
            stage_v[pl.ds(k * 16, 16)] = jnp.zeros((16,), jnp.float32)
            return carry
        lax.fori_loop(0, sl // 16, zbody, 0)
        pltpu.sync_copy(stage_v, dego_sh.at[pl.ds(off, sl)])
        pltpu.sync_copy(stage_v, degi_sh.at[pl.ds(off, sl)])
        pltpu.sync_copy(maskp_hbm.at[w], val_v)
        plsc.subcore_barrier()

        pltpu.sync_copy(srcp_hbm.at[w], idx_v)

        def sbody(j, carry):
            pltpu.sync_copy(val_v.at[j], dego_sh.at[idx_v.at[j]], add=True)
            return carry
        lax.fori_loop(0, nch, sbody, 0)

        pltpu.sync_copy(dstp_hbm.at[w], idx_v)

        def dbody(j, carry):
            pltpu.sync_copy(val_v.at[j], degi_sh.at[idx_v.at[j]], add=True)
            return carry
        lax.fori_loop(0, nch, dbody, 0)

        plsc.subcore_barrier()
        pltpu.sync_copy(dego_sh.at[pl.ds(off, sl)], stage_v)
        pltpu.sync_copy(stage_v, out_hbm.at[pl.ds(c * 2 * n_pad + off, sl)])
        pltpu.sync_copy(degi_sh.at[pl.ds(off, sl)], stage_v)
        pltpu.sync_copy(
            stage_v, out_hbm.at[pl.ds((c * 2 + 1) * n_pad + off, sl)])

    return deg_kernel(srcp, dstp, maskp)


def _sc_aggregate(srcp, dstp, hw, zrow, n_pad, sl):
    nw, nch, cc = srcp.shape
    dh = hw.shape[1]

    @functools.partial(
        pl.kernel,
        out_type=jax.ShapeDtypeStruct((_NC * n_pad, dh), jnp.float32),
        mesh=_sc_mesh(),
        compiler_params=_SC_PARAMS,
        scratch_types=[
            pltpu.VMEM((nch, cc), jnp.int32),
            pltpu.VMEM((nch, cc), jnp.int32),
            pltpu.VMEM((cc, dh), jnp.float32),
            pltpu.VMEM((cc, dh), jnp.float32),
            pltpu.VMEM_SHARED((n_pad, dh), jnp.float32),
            pltpu.SemaphoreType.DMA,
            pltpu.SemaphoreType.DMA,
        ],
    )
    def agg_kernel(srcp_hbm, dstp_hbm, hw_hbm, z_hbm, out_hbm,
                   src_v, dst_v, buf_a, buf_b, agg_sh, sem_a, sem_b):
        c = lax.axis_index("c")
        s = lax.axis_index("s")
        w = c * _NS + s
        off = s * sl

        pltpu.sync_copy(srcp_hbm.at[w], src_v)
        pltpu.sync_copy(dstp_hbm.at[w], dst_v)
        pltpu.sync_copy(z_hbm, buf_a)
        pltpu.sync_copy(buf_a, agg_sh.at[pl.ds(off, cc)])
        pltpu.sync_copy(buf_a.at[pl.ds(0, sl - cc)],
                        agg_sh.at[pl.ds(off + cc, sl - cc)])
        plsc.subcore_barrier()

        pltpu.async_copy(hw_hbm.at[src_v.at[0]], buf_a, sem_a)

        def body(t, carry):
            j = 2 * t
            pltpu.async_copy(hw_hbm.at[src_v.at[j + 1]], buf_b, sem_b)
            pltpu.make_async_copy(hw_hbm.at[src_v.at[j]], buf_a, sem_a).wait()
            pltpu.sync_copy(buf_a, agg_sh.at[dst_v.at[j]], add=True)

            @pl.when(t < nch // 2 - 1)
            def _():
                pltpu.async_copy(hw_hbm.at[src_v.at[j + 2]], buf_a, sem_a)

            pltpu.make_async_copy(
                hw_hbm.at[src_v.at[j + 1]], buf_b, sem_b).wait()
            pltpu.sync_copy(buf_b, agg_sh.at[dst_v.at[j + 1]], add=True)
            return carry
        lax.fori_loop(0, nch // 2, body, 0)

        plsc.subcore_barrier()
        pltpu.sync_copy(agg_sh.at[pl.ds(off, cc)], buf_a)
        pltpu.sync_copy(buf_a, out_hbm.at[pl.ds(c * n_pad + off, cc)])
        pltpu.sync_copy(agg_sh.at[pl.ds(off + cc, sl - cc)],
                        buf_a.at[pl.ds(0, sl - cc)])
        pltpu.sync_copy(buf_a.at[pl.ds(0, sl - cc)],
                        out_hbm.at[pl.ds(c * n_pad + off + cc, sl - cc)])

    return agg_kernel(srcp, dstp, hw, zrow)


def _tc_project(feats, deg4, w_mat, bn):
    n, di = feats.shape
    dh = w_mat.shape[1]

    def body(feats_ref, deg_ref, w_ref, hw_ref, nd_ref):
        deg_o = deg_ref[:, 0:1] + deg_ref[:, 2:3]
        deg_i = deg_ref[:, 1:2] + deg_ref[:, 3:4]
        norm_o = lax.rsqrt(jnp.maximum(deg_o, 1.0))
        nd_ref[...] = lax.rsqrt(jnp.maximum(deg_i, 1.0))
        h = feats_ref[...] * norm_o
        hw_ref[...] = jnp.dot(h, w_ref[...],
                              preferred_element_type=jnp.float32)

    return pl.pallas_call(
        body,
        grid=(n // bn,),
        in_specs=[
            pl.BlockSpec((bn, di), lambda i: (i, 0)),
            pl.BlockSpec((bn, 4), lambda i: (i, 0)),
            pl.BlockSpec((di, dh), lambda i: (0, 0)),
        ],
        out_specs=[
            pl.BlockSpec((bn, dh), lambda i: (i, 0)),
            pl.BlockSpec((bn, 1), lambda i: (i, 0)),
        ],
        out_shape=[
            jax.ShapeDtypeStruct((n, dh), jnp.float32),
            jax.ShapeDtypeStruct((n, 1), jnp.float32),
        ],
    )(feats, deg4, w_mat)


def _tc_finish(agg0, agg1, norm_dst, b2, bn):
    n, dh = agg0.shape

    def body(a0_ref, a1_ref, nd_ref, b_ref, out_ref):
        acc = (a0_ref[...] + a1_ref[...]) * nd_ref[...]
        out_ref[...] = jnp.maximum(acc + b_ref[...], 0.0)

    return pl.pallas_call(
        body,
        grid=(n // bn,),
        in_specs=[
            pl.BlockSpec((bn, dh), lambda i: (i, 0)),
            pl.BlockSpec((bn, dh), lambda i: (i, 0)),
            pl.BlockSpec((bn, 1), lambda i: (i, 0)),
            pl.BlockSpec((1, dh), lambda i: (0, 0)),
        ],
        out_specs=pl.BlockSpec((bn, dh), lambda i: (i, 0)),
        out_shape=jax.ShapeDtypeStruct((n, dh), jnp.float32),
    )(agg0, agg1, norm_dst, b2)


def kernel(feats, edge_index, W, b):
    n, di = feats.shape
    dh = W.shape[1]
    e = edge_index.shape[1]
    nw = _NC * _NS

    cc = 512
    ept = -(-e // (nw * 2 * cc)) * 2 * cc
    nch = ept // cc
    e_pad = nw * ept
    pad = e_pad - e
    sl = -(-(n + 1) // (_NS * _L)) * _L
    n_pad = _NS * sl

    src = edge_index[0]
    dst = edge_index[1]
    srcp = jnp.concatenate(
        [src, jnp.zeros((pad,), jnp.int32)]).reshape(nw, nch, cc)
    dstp = jnp.concatenate(
        [dst, jnp.full((pad,), n, jnp.int32)]).reshape(nw, nch, cc)
    maskp = jnp.concatenate(
        [jnp.ones((e,), jnp.float32),
         jnp.zeros((pad,), jnp.float32)]).reshape(nw, nch, cc)

    deg_part = _sc_degrees(srcp, dstp, maskp, n_pad, sl).reshape(4, n_pad)
    deg4 = deg_part[:, :n].T

    bn = 2000 if n % 2000 == 0 else n
    hw, norm_dst = _tc_project(feats, deg4, W, bn)

    zrow = jnp.zeros((cc, dh), jnp.float32)
    agg_part = _sc_aggregate(srcp, dstp, hw, zrow, n_pad, sl)
    agg_part = agg_part.reshape(_NC, n_pad, dh)

    return _tc_finish(agg_part[0, :n], agg_part[1, :n], norm_dst,
                      b.reshape(1, dh), bn)


# speedup vs baseline: 1.9048x; 1.9048x over previous
import functools

import jax
import jax.numpy as jnp
from jax import lax
from jax.experimental import pallas as pl
from jax.experimental.pallas import tpu as pltpu
from jax.experimental.pallas import tpu_sc as plsc

_NC = 2
_NS = 16
_L = 128


def _sc_mesh():
    return plsc.VectorSubcoreMesh(
        core_axis_name="c", subcore_axis_name="s",
        num_cores=_NC, num_subcores=_NS)


_SC_PARAMS = pltpu.CompilerParams(use_tc_tiling_on_sc=False)


def _sc_degrees(srcp, dstp, maskp, n_pad, sl):
    nw, nch, cc = srcp.shape

    @functools.partial(
        pl.kernel,
        out_type=jax.ShapeDtypeStruct((_NC * 2 * n_pad,), jnp.float32),
        mesh=_sc_mesh(),
        compiler_params=_SC_PARAMS,
        scratch_types=[
            pltpu.VMEM((nch, cc), jnp.int32),
            pltpu.VMEM((nch, cc), jnp.float32),
            pltpu.VMEM((sl,), jnp.float32),
            pltpu.VMEM_SHARED((n_pad,), jnp.float32),
            pltpu.VMEM_SHARED((n_pad,), jnp.float32),
        ],
    )
    def deg_kernel(srcp_hbm, dstp_hbm, maskp_hbm, out_hbm,
                   idx_v, val_v, stage_v, dego_sh, degi_sh):
        c = lax.axis_index("c")
        s = lax.axis_index("s")
        w = c * _NS + s
        off = s * sl

        def zbody(k, carry):
            stage_v[pl.ds(k * 16, 16)] = jnp.zeros((16,), jnp.float32)
            return carry
        lax.fori_loop(0, sl // 16, zbody, 0)
        pltpu.sync_copy(stage_v, dego_sh.at[pl.ds(off, sl)])
        pltpu.sync_copy(stage_v, degi_sh.at[pl.ds(off, sl)])
        pltpu.sync_copy(maskp_hbm.at[w], val_v)
        plsc.subcore_barrier()

        pltpu.sync_copy(srcp_hbm.at[w], idx_v)

        def sbody(j, carry):
            pltpu.sync_copy(val_v.at[j], dego_sh.at[idx_v.at[j]], add=True)
            return carry
        lax.fori_loop(0, nch, sbody, 0)

        pltpu.sync_copy(dstp_hbm.at[w], idx_v)

        def dbody(j, carry):
            pltpu.sync_copy(val_v.at[j], degi_sh.at[idx_v.at[j]], add=True)
            return carry
        lax.fori_loop(0, nch, dbody, 0)

        plsc.subcore_barrier()
        pltpu.sync_copy(dego_sh.at[pl.ds(off, sl)], stage_v)
        pltpu.sync_copy(stage_v, out_hbm.at[pl.ds(c * 2 * n_pad + off, sl)])
        pltpu.sync_copy(degi_sh.at[pl.ds(off, sl)], stage_v)
        pltpu.sync_copy(
            stage_v, out_hbm.at[pl.ds((c * 2 + 1) * n_pad + off, sl)])

    return deg_kernel(srcp, dstp, maskp)


def _sc_aggregate(srcp, dstp, hw, zrow, n_pad, sl):
    nw, nch, cc = srcp.shape
    dh = hw.shape[1]

    @functools.partial(
        pl.kernel,
        out_type=jax.ShapeDtypeStruct((_NC * n_pad, dh), jnp.float32),
        mesh=_sc_mesh(),
        compiler_params=_SC_PARAMS,
        scratch_types=[
            pltpu.VMEM((nch, cc), jnp.int32),
            pltpu.VMEM((nch, cc), jnp.int32),
            pltpu.VMEM((cc, dh), jnp.float32),
            pltpu.VMEM((cc, dh), jnp.float32),
            pltpu.VMEM_SHARED((n_pad, dh), jnp.float32),
            pltpu.VMEM_SHARED((n_pad, dh), jnp.float32),
            pltpu.SemaphoreType.DMA,
            pltpu.SemaphoreType.DMA,
        ],
    )
    def agg_kernel(srcp_hbm, dstp_hbm, hw_hbm, z_hbm, out_hbm,
                   src_v, dst_v, buf_a, buf_b, tab_sh, agg_sh, sem_a, sem_b):
        c = lax.axis_index("c")
        s = lax.axis_index("s")
        w = c * _NS + s
        off = s * sl
        nz = sl // cc

        pltpu.sync_copy(srcp_hbm.at[w], src_v)
        pltpu.sync_copy(dstp_hbm.at[w], dst_v)
        pltpu.sync_copy(z_hbm, buf_a)

        def zb(k, carry):
            pltpu.sync_copy(buf_a, agg_sh.at[pl.ds(off + k * cc, cc)])
            return carry
        lax.fori_loop(0, nz, zb, 0)

        def tb(k, carry):
            pltpu.sync_copy(hw_hbm.at[pl.ds(off + k * cc, cc)], buf_b)
            pltpu.sync_copy(buf_b, tab_sh.at[pl.ds(off + k * cc, cc)])
            return carry
        lax.fori_loop(0, nz, tb, 0)
        plsc.subcore_barrier()

        pltpu.async_copy(tab_sh.at[src_v.at[0]], buf_a, sem_a)

        def body(t, carry):
            j = 2 * t
            pltpu.async_copy(tab_sh.at[src_v.at[j + 1]], buf_b, sem_b)
            pltpu.make_async_copy(tab_sh.at[src_v.at[j]], buf_a, sem_a).wait()
            pltpu.sync_copy(buf_a, agg_sh.at[dst_v.at[j]], add=True)

            @pl.when(t < nch // 2 - 1)
            def _():
                pltpu.async_copy(tab_sh.at[src_v.at[j + 2]], buf_a, sem_a)

            pltpu.make_async_copy(
                tab_sh.at[src_v.at[j + 1]], buf_b, sem_b).wait()
            pltpu.sync_copy(buf_b, agg_sh.at[dst_v.at[j + 1]], add=True)
            return carry
        lax.fori_loop(0, nch // 2, body, 0)

        plsc.subcore_barrier()

        def ob(k, carry):
            pltpu.sync_copy(agg_sh.at[pl.ds(off + k * cc, cc)], buf_a)
            pltpu.sync_copy(
                buf_a, out_hbm.at[pl.ds(c * n_pad + off + k * cc, cc)])
            return carry
        lax.fori_loop(0, nz, ob, 0)

    return agg_kernel(srcp, dstp, hw, zrow)


def _tc_project(feats, deg4, w_mat, bn):
    n, di = feats.shape
    dh = w_mat.shape[1]

    def body(feats_ref, deg_ref, w_ref, hw_ref, nd_ref):
        deg_o = deg_ref[:, 0:1] + deg_ref[:, 2:3]
        deg_i = deg_ref[:, 1:2] + deg_ref[:, 3:4]
        norm_o = lax.rsqrt(jnp.maximum(deg_o, 1.0))
        nd_ref[...] = lax.rsqrt(jnp.maximum(deg_i, 1.0))
        h = feats_ref[...] * norm_o
        hw_ref[...] = jnp.dot(h, w_ref[...],
                              preferred_element_type=jnp.float32)

    return pl.pallas_call(
        body,
        grid=(n // bn,),
        in_specs=[
            pl.BlockSpec((bn, di), lambda i: (i, 0)),
            pl.BlockSpec((bn, 4), lambda i: (i, 0)),
            pl.BlockSpec((di, dh), lambda i: (0, 0)),
        ],
        out_specs=[
            pl.BlockSpec((bn, dh), lambda i: (i, 0)),
            pl.BlockSpec((bn, 1), lambda i: (i, 0)),
        ],
        out_shape=[
            jax.ShapeDtypeStruct((n, dh), jnp.float32),
            jax.ShapeDtypeStruct((n, 1), jnp.float32),
        ],
    )(feats, deg4, w_mat)


def _tc_finish(agg0, agg1, norm_dst, b2, bn):
    n, dh = agg0.shape

    def body(a0_ref, a1_ref, nd_ref, b_ref, out_ref):
        acc = (a0_ref[...] + a1_ref[...]) * nd_ref[...]
        out_ref[...] = jnp.maximum(acc + b_ref[...], 0.0)

    return pl.pallas_call(
        body,
        grid=(n // bn,),
        in_specs=[
            pl.BlockSpec((bn, dh), lambda i: (i, 0)),
            pl.BlockSpec((bn, dh), lambda i: (i, 0)),
            pl.BlockSpec((bn, 1), lambda i: (i, 0)),
            pl.BlockSpec((1, dh), lambda i: (0, 0)),
        ],
        out_specs=pl.BlockSpec((bn, dh), lambda i: (i, 0)),
        out_shape=jax.ShapeDtypeStruct((n, dh), jnp.float32),
    )(agg0, agg1, norm_dst, b2)


def kernel(feats, edge_index, W, b):
    n, di = feats.shape
    dh = W.shape[1]
    e = edge_index.shape[1]
    nw = _NC * _NS

    cc = 128
    ept = -(-e // (nw * 2 * cc)) * 2 * cc
    nch = ept // cc
    e_pad = nw * ept
    pad = e_pad - e
    sl = -(-(n + 1) // (_NS * _L)) * _L
    n_pad = _NS * sl

    src = edge_index[0]
    dst = edge_index[1]
    srcp = jnp.concatenate(
        [src, jnp.zeros((pad,), jnp.int32)]).reshape(nw, nch, cc)
    dstp = jnp.concatenate(
        [dst, jnp.full((pad,), n, jnp.int32)]).reshape(nw, nch, cc)
    maskp = jnp.concatenate(
        [jnp.ones((e,), jnp.float32),
         jnp.zeros((pad,), jnp.float32)]).reshape(nw, nch, cc)

    deg_part = _sc_degrees(srcp, dstp, maskp, n_pad, sl).reshape(4, n_pad)
    deg4 = deg_part[:, :n].T

    bn = 2000 if n % 2000 == 0 else n
    hw, norm_dst = _tc_project(feats, deg4, W, bn)

    hw_pad = jnp.pad(hw, ((0, n_pad - n), (0, 0)))
    zrow = jnp.zeros((cc, dh), jnp.float32)
    agg_part = _sc_aggregate(srcp, dstp, hw_pad, zrow, n_pad, sl)
    agg_part = agg_part.reshape(_NC, n_pad, dh)

    return _tc_finish(agg_part[0, :n], agg_part[1, :n], norm_dst,
                      b.reshape(1, dh), bn)
